# baseline (speedup 1.0000x reference)
import functools

import jax
import jax.numpy as jnp
from jax import lax
from jax.experimental import pallas as pl
from jax.experimental.pallas import tpu as pltpu
from jax.experimental.pallas import tpu_sc as plsc

V = 100000
D = 32
WIN = 20
HID = 32
B = 1024
NTOK = B * WIN
FEAT = WIN * D

VB = 2048
NV = (V + VB - 1) // VB


def _sc_gather(emb, idx):
    info = plsc.get_sparse_core_info()
    nc, ns = info.num_cores, info.num_subcores
    nw = nc * ns
    bpw = NTOK // nw

    mesh = plsc.VectorSubcoreMesh(core_axis_name="c", subcore_axis_name="s")

    @functools.partial(
        pl.kernel,
        mesh=mesh,
        compiler_params=pltpu.CompilerParams(use_tc_tiling_on_sc=False),
        out_type=jax.ShapeDtypeStruct((NTOK, D), jnp.float32),
        scratch_types=[
            pltpu.VMEM((bpw,), jnp.int32),
            pltpu.VMEM((bpw, D), jnp.float32),
            pltpu.SemaphoreType.DMA,
        ],
    )
    def gather(table_hbm, idx_hbm, out_hbm, idx_v, rows_v, sem):
        wid = lax.axis_index("s") * nc + lax.axis_index("c")
        base = wid * bpw
        pltpu.sync_copy(idx_hbm.at[pl.ds(base, bpw)], idx_v)
        pltpu.async_copy(table_hbm.at[idx_v], rows_v, sem).wait()
        pltpu.sync_copy(rows_v, out_hbm.at[pl.ds(base, bpw)])

    return gather(emb, idx)


def _stats_body(h0_ref, w1_ref, b1_ref, w2_ref, b2_ref, h_ref, m_ref, s_ref):
    v = pl.program_id(0)

    @pl.when(v == 0)
    def _init():
        h = lax.dot_general(h0_ref[...], w1_ref[...], (((1,), (1,)), ((), ())),
                            preferred_element_type=jnp.float32)
        h_ref[...] = jnp.tanh(h + b1_ref[...])
        m_ref[...] = jnp.full((B, 1), -jnp.inf, jnp.float32)
        s_ref[...] = jnp.zeros((B, 1), jnp.float32)

    logits = lax.dot_general(h_ref[...], w2_ref[...], (((1,), (1,)), ((), ())),
                             preferred_element_type=jnp.float32) + b2_ref[...]
    col = lax.broadcasted_iota(jnp.int32, (B, VB), 1)
    logits = jnp.where(col < (V - v * VB), logits, -jnp.inf)
    m_old = m_ref[...]
    m_new = jnp.maximum(m_old, jnp.max(logits, axis=1, keepdims=True))
    s_ref[...] = (s_ref[...] * jnp.exp(m_old - m_new)
                  + jnp.sum(jnp.exp(logits - m_new), axis=1, keepdims=True))
    m_ref[...] = m_new


def _out_body(h_ref, m_ref, s_ref, w2_ref, b2_ref, o_ref):
    logits = lax.dot_general(h_ref[...], w2_ref[...], (((1,), (1,)), ((), ())),
                             preferred_element_type=jnp.float32) + b2_ref[...]
    o_ref[...] = jnp.exp(logits - m_ref[...]) * (1.0 / s_ref[...])


def _dense(h0, W1, b1, W2, b2):
    b1r = b1.reshape(1, HID)
    b2r = b2.reshape(1, V)

    h, m, s = pl.pallas_call(
        _stats_body,
        grid=(NV,),
        in_specs=[
            pl.BlockSpec((B, FEAT), lambda v: (0, 0)),
            pl.BlockSpec((HID, FEAT), lambda v: (0, 0)),
            pl.BlockSpec((1, HID), lambda v: (0, 0)),
            pl.BlockSpec((VB, HID), lambda v: (v, 0)),
            pl.BlockSpec((1, VB), lambda v: (0, v)),
        ],
        out_specs=[
            pl.BlockSpec((B, HID), lambda v: (0, 0)),
            pl.BlockSpec((B, 1), lambda v: (0, 0)),
            pl.BlockSpec((B, 1), lambda v: (0, 0)),
        ],
        out_shape=[
            jax.ShapeDtypeStruct((B, HID), jnp.float32),
            jax.ShapeDtypeStruct((B, 1), jnp.float32),
            jax.ShapeDtypeStruct((B, 1), jnp.float32),
        ],
    )(h0, W1, b1r, W2, b2r)

    out = pl.pallas_call(
        _out_body,
        grid=(NV,),
        in_specs=[
            pl.BlockSpec((B, HID), lambda v: (0, 0)),
            pl.BlockSpec((B, 1), lambda v: (0, 0)),
            pl.BlockSpec((B, 1), lambda v: (0, 0)),
            pl.BlockSpec((VB, HID), lambda v: (v, 0)),
            pl.BlockSpec((1, VB), lambda v: (0, v)),
        ],
        out_specs=pl.BlockSpec((B, VB), lambda v: (0, v)),
        out_shape=jax.ShapeDtypeStruct((B, V), jnp.float32),
    )(h, m, s, W2, b2r)
    return out


def kernel(x, emb, W1, b1, W2, b2):
    h0 = _sc_gather(emb, x.reshape(-1)).reshape(B, FEAT)
    return _dense(h0, W1, b1, W2, b2)


# speedup vs baseline: 1.0579x; 1.0579x over previous
import functools

import jax
import jax.numpy as jnp
from jax import lax
from jax.experimental import pallas as pl
from jax.experimental.pallas import tpu as pltpu
from jax.experimental.pallas import tpu_sc as plsc

V = 100000
D = 32
WIN = 20
HID = 32
B = 1024
NTOK = B * WIN
FEAT = WIN * D

VB = 2048
NV = (V + VB - 1) // VB


def _sc_gather(emb, idx):
    info = plsc.get_sparse_core_info()
    nc, ns = info.num_cores, info.num_subcores
    nw = nc * ns
    bpw = NTOK // nw

    mesh = plsc.VectorSubcoreMesh(core_axis_name="c", subcore_axis_name="s")

    @functools.partial(
        pl.kernel,
        mesh=mesh,
        compiler_params=pltpu.CompilerParams(use_tc_tiling_on_sc=False),
        out_type=jax.ShapeDtypeStruct((NTOK, D), jnp.float32),
        scratch_types=[
            pltpu.VMEM((bpw,), jnp.int32),
            pltpu.VMEM((bpw, D), jnp.float32),
            pltpu.SemaphoreType.DMA,
        ],
    )
    def gather(table_hbm, idx_hbm, out_hbm, idx_v, rows_v, sem):
        wid = lax.axis_index("s") * nc + lax.axis_index("c")
        base = wid * bpw
        pltpu.sync_copy(idx_hbm.at[pl.ds(base, bpw)], idx_v)
        pltpu.async_copy(table_hbm.at[idx_v], rows_v, sem).wait()
        pltpu.sync_copy(rows_v, out_hbm.at[pl.ds(base, bpw)])

    return gather(emb, idx)


def _stats_body(h0_ref, w1_ref, b1_ref, w2_ref, b2_ref, h_ref, m_ref, s_ref):
    v = pl.program_id(0)

    @pl.when(v == 0)
    def _init():
        h = lax.dot_general(h0_ref[...], w1_ref[...], (((1,), (1,)), ((), ())),
                            preferred_element_type=jnp.float32)
        h_ref[...] = jnp.tanh(h + b1_ref[...]).astype(jnp.bfloat16)
        m_ref[...] = jnp.full((B, 1), -jnp.inf, jnp.float32)
        s_ref[...] = jnp.zeros((B, 1), jnp.float32)

    logits = lax.dot_general(h_ref[...], w2_ref[...], (((1,), (0,)), ((), ())),
                             preferred_element_type=jnp.float32) + b2_ref[...]
    col = lax.broadcasted_iota(jnp.int32, (B, VB), 1)
    logits = jnp.where(col < (V - v * VB), logits, -jnp.inf)
    m_old = m_ref[...]
    m_new = jnp.maximum(m_old, jnp.max(logits, axis=1, keepdims=True))
    s_ref[...] = (s_ref[...] * jnp.exp(m_old - m_new)
                  + jnp.sum(jnp.exp(logits - m_new), axis=1, keepdims=True))
    m_ref[...] = m_new


def _out_body(h_ref, m_ref, s_ref, w2_ref, b2_ref, o_ref):
    logits = lax.dot_general(h_ref[...], w2_ref[...], (((1,), (0,)), ((), ())),
                             preferred_element_type=jnp.float32) + b2_ref[...]
    o_ref[...] = jnp.exp(logits - m_ref[...]) * (1.0 / s_ref[...])


def _dense(h0, W1, b1, W2, b2):
    b1r = b1.reshape(1, HID)
    b2r = b2.reshape(1, V)
    W2t = W2.T.astype(jnp.bfloat16)

    h, m, s = pl.pallas_call(
        _stats_body,
        grid=(NV,),
        in_specs=[
            pl.BlockSpec((B, FEAT), lambda v: (0, 0)),
            pl.BlockSpec((HID, FEAT), lambda v: (0, 0)),
            pl.BlockSpec((1, HID), lambda v: (0, 0)),
            pl.BlockSpec((HID, VB), lambda v: (0, v)),
            pl.BlockSpec((1, VB), lambda v: (0, v)),
        ],
        out_specs=[
            pl.BlockSpec((B, HID), lambda v: (0, 0)),
            pl.BlockSpec((B, 1), lambda v: (0, 0)),
            pl.BlockSpec((B, 1), lambda v: (0, 0)),
        ],
        out_shape=[
            jax.ShapeDtypeStruct((B, HID), jnp.bfloat16),
            jax.ShapeDtypeStruct((B, 1), jnp.float32),
            jax.ShapeDtypeStruct((B, 1), jnp.float32),
        ],
    )(h0, W1, b1r, W2t, b2r)

    out = pl.pallas_call(
        _out_body,
        grid=(NV,),
        in_specs=[
            pl.BlockSpec((B, HID), lambda v: (0, 0)),
            pl.BlockSpec((B, 1), lambda v: (0, 0)),
            pl.BlockSpec((B, 1), lambda v: (0, 0)),
            pl.BlockSpec((HID, VB), lambda v: (0, v)),
            pl.BlockSpec((1, VB), lambda v: (0, v)),
        ],
        out_specs=pl.BlockSpec((B, VB), lambda v: (0, v)),
        out_shape=jax.ShapeDtypeStruct((B, V), jnp.float32),
    )(h, m, s, W2t, b2r)
    return out


def kernel(x, emb, W1, b1, W2, b2):
    h0 = _sc_gather(emb, x.reshape(-1)).reshape(B, FEAT)
    return _dense(h0, W1, b1, W2, b2)


# speedup vs baseline: 1.0664x; 1.0080x over previous
import functools

import jax
import jax.numpy as jnp
from jax import lax
from jax.experimental import pallas as pl
from jax.experimental.pallas import tpu as pltpu
from jax.experimental.pallas import tpu_sc as plsc

V = 100000
D = 32
WIN = 20
HID = 32
B = 1024
NTOK = B * WIN
FEAT = WIN * D

VB = 4096
NV = (V + VB - 1) // VB
VP = NV * VB


def _sc_gather(emb, idx):
    info = plsc.get_sparse_core_info()
    nc, ns = info.num_cores, info.num_subcores
    nw = nc * ns
    bpw = NTOK // nw

    mesh = plsc.VectorSubcoreMesh(core_axis_name="c", subcore_axis_name="s")

    @functools.partial(
        pl.kernel,
        mesh=mesh,
        compiler_params=pltpu.CompilerParams(use_tc_tiling_on_sc=False),
        out_type=jax.ShapeDtypeStruct((NTOK, D), jnp.float32),
        scratch_types=[
            pltpu.VMEM((bpw,), jnp.int32),
            pltpu.VMEM((bpw, D), jnp.float32),
            pltpu.SemaphoreType.DMA,
        ],
    )
    def gather(table_hbm, idx_hbm, out_hbm, idx_v, rows_v, sem):
        wid = lax.axis_index("s") * nc + lax.axis_index("c")
        base = wid * bpw
        pltpu.sync_copy(idx_hbm.at[pl.ds(base, bpw)], idx_v)
        pltpu.async_copy(table_hbm.at[idx_v], rows_v, sem).wait()
        pltpu.sync_copy(rows_v, out_hbm.at[pl.ds(base, bpw)])

    return gather(emb, idx)


def _stats_body(h0_ref, w1_ref, b1_ref, w2_ref, b2_ref, h_ref, z_ref,
                m_ref, s_ref):
    v = pl.program_id(0)

    @pl.when(v == 0)
    def _init():
        h = lax.dot_general(h0_ref[...], w1_ref[...], (((1,), (1,)), ((), ())),
                            preferred_element_type=jnp.float32)
        h_ref[...] = jnp.tanh(h + b1_ref[...]).astype(jnp.bfloat16)
        m_ref[...] = jnp.full((B, 1), -jnp.inf, jnp.float32)
        s_ref[...] = jnp.zeros((B, 1), jnp.float32)

    logits = lax.dot_general(h_ref[...], w2_ref[...], (((1,), (0,)), ((), ())),
                             preferred_element_type=jnp.float32) + b2_ref[...]
    m_old = m_ref[...]
    m_new = jnp.maximum(m_old, jnp.max(logits, axis=1, keepdims=True))
    s_ref[...] = (s_ref[...] * jnp.exp(m_old - m_new)
                  + jnp.sum(jnp.exp(logits - m_new), axis=1, keepdims=True))
    m_ref[...] = m_new

    @pl.when(v == NV - 1)
    def _fin():
        z_ref[...] = m_ref[...] + jnp.log(s_ref[...])


def _out_body(h_ref, z_ref, w2_ref, b2_ref, o_ref):
    logits = lax.dot_general(h_ref[...], w2_ref[...], (((1,), (0,)), ((), ())),
                             preferred_element_type=jnp.float32) + b2_ref[...]
    o_ref[...] = jnp.exp(logits - z_ref[...])


def _dense(h0, W1, b1, W2, b2):
    b1r = b1.reshape(1, HID)
    W2t = jnp.pad(W2.T.astype(jnp.bfloat16), ((0, 0), (0, VP - V)))
    b2r = jnp.pad(b2.reshape(1, V), ((0, 0), (0, VP - V)),
                  constant_values=-1e30)

    h, z, _, _ = pl.pallas_call(
        _stats_body,
        grid=(NV,),
        in_specs=[
            pl.BlockSpec((B, FEAT), lambda v: (0, 0)),
            pl.BlockSpec((HID, FEAT), lambda v: (0, 0)),
            pl.BlockSpec((1, HID), lambda v: (0, 0)),
            pl.BlockSpec((HID, VB), lambda v: (0, v)),
            pl.BlockSpec((1, VB), lambda v: (0, v)),
        ],
        out_specs=[
            pl.BlockSpec((B, HID), lambda v: (0, 0)),
            pl.BlockSpec((B, 1), lambda v: (0, 0)),
            pl.BlockSpec((B, 1), lambda v: (0, 0)),
            pl.BlockSpec((B, 1), lambda v: (0, 0)),
        ],
        out_shape=[
            jax.ShapeDtypeStruct((B, HID), jnp.bfloat16),
            jax.ShapeDtypeStruct((B, 1), jnp.float32),
            jax.ShapeDtypeStruct((B, 1), jnp.float32),
            jax.ShapeDtypeStruct((B, 1), jnp.float32),
        ],
    )(h0, W1, b1r, W2t, b2r)

    out = pl.pallas_call(
        _out_body,
        grid=(NV,),
        in_specs=[
            pl.BlockSpec((B, HID), lambda v: (0, 0)),
            pl.BlockSpec((B, 1), lambda v: (0, 0)),
            pl.BlockSpec((HID, VB), lambda v: (0, v)),
            pl.BlockSpec((1, VB), lambda v: (0, v)),
        ],
        out_specs=pl.BlockSpec((B, VB), lambda v: (0, v)),
        out_shape=jax.ShapeDtypeStruct((B, V), jnp.float32),
    )(h, z, W2t, b2r)
    return out


def kernel(x, emb, W1, b1, W2, b2):
    h0 = _sc_gather(emb, x.reshape(-1)).reshape(B, FEAT)
    return _dense(h0, W1, b1, W2, b2)


# speedup vs baseline: 1.0692x; 1.0027x over previous
import functools

import jax
import jax.numpy as jnp
from jax import lax
from jax.experimental import pallas as pl
from jax.experimental.pallas import tpu as pltpu
from jax.experimental.pallas import tpu_sc as plsc

V = 100000
D = 32
WIN = 20
HID = 32
B = 1024
NTOK = B * WIN
FEAT = WIN * D

VB = 4096
NV = (V + VB - 1) // VB
VP = NV * VB
CH = 256
NCH = VB // CH
KA = 48


def _sc_gather(emb, idx):
    info = plsc.get_sparse_core_info()
    nc, ns = info.num_cores, info.num_subcores
    nw = nc * ns
    bpw = NTOK // nw

    mesh = plsc.VectorSubcoreMesh(core_axis_name="c", subcore_axis_name="s")

    @functools.partial(
        pl.kernel,
        mesh=mesh,
        compiler_params=pltpu.CompilerParams(use_tc_tiling_on_sc=False),
        out_type=jax.ShapeDtypeStruct((NTOK, D), jnp.float32),
        scratch_types=[
            pltpu.VMEM((bpw,), jnp.int32),
            pltpu.VMEM((bpw, D), jnp.float32),
            pltpu.SemaphoreType.DMA,
        ],
    )
    def gather(table_hbm, idx_hbm, out_hbm, idx_v, rows_v, sem):
        wid = lax.axis_index("s") * nc + lax.axis_index("c")
        base = wid * bpw
        pltpu.sync_copy(idx_hbm.at[pl.ds(base, bpw)], idx_v)
        pltpu.async_copy(table_hbm.at[idx_v], rows_v, sem).wait()
        pltpu.sync_copy(rows_v, out_hbm.at[pl.ds(base, bpw)])

    return gather(emb, idx)


def _stats_body(h0_ref, w1_ref, b1_ref, w2_ref, b2_ref, h_ref, z_ref,
                m_ref, s_ref):
    v = pl.program_id(0)

    @pl.when(v == 0)
    def _init():
        h = lax.dot_general(h0_ref[...], w1_ref[...], (((1,), (1,)), ((), ())),
                            preferred_element_type=jnp.float32)
        h_ref[...] = jnp.tanh(h + b1_ref[...]).astype(jnp.bfloat16)
        m_ref[...] = jnp.full((B, 1), -jnp.inf, jnp.float32)
        s_ref[...] = jnp.zeros((B, 1), jnp.float32)

    logits = lax.dot_general(h_ref[...], w2_ref[...], (((1,), (0,)), ((), ())),
                             preferred_element_type=jnp.float32) + b2_ref[...]
    m_old = m_ref[...]
    m_new = jnp.maximum(m_old, jnp.max(logits, axis=1, keepdims=True))
    s_ref[...] = (s_ref[...] * jnp.exp(m_old - m_new)
                  + jnp.sum(jnp.exp(logits - m_new), axis=1, keepdims=True))
    m_ref[...] = m_new

    @pl.when(v == NV - 1)
    def _fin():
        z_ref[...] = m_ref[...] + jnp.log(s_ref[...])


def _out_body(h_ref, z_ref, w2_ref, b2_ref, o_ref):
    logits = lax.dot_general(h_ref[...], w2_ref[...], (((1,), (0,)), ((), ())),
                             preferred_element_type=jnp.float32) + b2_ref[...]
    o_ref[...] = jnp.exp(logits - z_ref[...])


def _dense(h0, W1, b1, W2, b2):
    b1r = b1.reshape(1, HID)
    W2t = jnp.pad(W2.T.astype(jnp.bfloat16), ((0, 0), (0, VP - V)))
    b2r = jnp.pad(b2.reshape(1, V), ((0, 0), (0, VP - V)),
                  constant_values=-1e30)

    h, z, _, _ = pl.pallas_call(
        _stats_body,
        grid=(NV,),
        in_specs=[
            pl.BlockSpec((B, FEAT), lambda v: (0, 0)),
            pl.BlockSpec((HID, FEAT), lambda v: (0, 0)),
            pl.BlockSpec((1, HID), lambda v: (0, 0)),
            pl.BlockSpec((HID, VB), lambda v: (0, v)),
            pl.BlockSpec((1, VB), lambda v: (0, v)),
        ],
        out_specs=[
            pl.BlockSpec((B, HID), lambda v: (0, 0)),
            pl.BlockSpec((B, 1), lambda v: (0, 0)),
            pl.BlockSpec((B, 1), lambda v: (0, 0)),
            pl.BlockSpec((B, 1), lambda v: (0, 0)),
        ],
        out_shape=[
            jax.ShapeDtypeStruct((B, HID), jnp.bfloat16),
            jax.ShapeDtypeStruct((B, 1), jnp.float32),
            jax.ShapeDtypeStruct((B, 1), jnp.float32),
            jax.ShapeDtypeStruct((B, 1), jnp.float32),
        ],
    )(h0, W1, b1r, W2t, b2r)

    out = pl.pallas_call(
        _out_body,
        grid=(NV,),
        in_specs=[
            pl.BlockSpec((B, HID), lambda v: (0, 0)),
            pl.BlockSpec((B, 1), lambda v: (0, 0)),
            pl.BlockSpec((HID, VB), lambda v: (0, v)),
            pl.BlockSpec((1, VB), lambda v: (0, v)),
        ],
        out_specs=pl.BlockSpec((B, VB), lambda v: (0, v)),
        out_shape=jax.ShapeDtypeStruct((B, V), jnp.float32),
    )(h, z, W2t, b2r)
    return out


def kernel(x, emb, W1, b1, W2, b2):
    h0 = _sc_gather(emb, x.reshape(-1)).reshape(B, FEAT)
    return _dense(h0, W1, b1, W2, b2)


# speedup vs baseline: 1.0976x; 1.0265x over previous
import functools

import jax
import jax.numpy as jnp
from jax import lax
from jax.experimental import pallas as pl
from jax.experimental.pallas import tpu as pltpu
from jax.experimental.pallas import tpu_sc as plsc

V = 100000
D = 32
WIN = 20
HID = 32
B = 1024
NTOK = B * WIN
FEAT = WIN * D

VB = 4096
NV = (V + VB - 1) // VB
VP = NV * VB
CH = 256
NCH = VB // CH
KA = 48


def _sc_gather(emb, idx):
    info = plsc.get_sparse_core_info()
    nc, ns = info.num_cores, info.num_subcores
    nw = nc * ns
    bpw = NTOK // nw

    mesh = plsc.VectorSubcoreMesh(core_axis_name="c", subcore_axis_name="s")

    @functools.partial(
        pl.kernel,
        mesh=mesh,
        compiler_params=pltpu.CompilerParams(use_tc_tiling_on_sc=False),
        out_type=jax.ShapeDtypeStruct((NTOK, D), jnp.float32),
        scratch_types=[
            pltpu.VMEM((bpw,), jnp.int32),
            pltpu.VMEM((bpw, D), jnp.float32),
            pltpu.SemaphoreType.DMA,
        ],
    )
    def gather(table_hbm, idx_hbm, out_hbm, idx_v, rows_v, sem):
        wid = lax.axis_index("s") * nc + lax.axis_index("c")
        base = wid * bpw
        pltpu.sync_copy(idx_hbm.at[pl.ds(base, bpw)], idx_v)
        pltpu.async_copy(table_hbm.at[idx_v], rows_v, sem).wait()
        pltpu.sync_copy(rows_v, out_hbm.at[pl.ds(base, bpw)])

    return gather(emb, idx)


def _stats_body(h0_ref, w1_ref, b1_ref, w2a_ref, h_ref, z_ref, m_ref,
                hn_ref, acc_ref):
    v = pl.program_id(0)

    @pl.when(v == 0)
    def _init():
        h = lax.dot_general(h0_ref[...], w1_ref[...], (((1,), (1,)), ((), ())),
                            preferred_element_type=jnp.float32)
        h = jnp.tanh(h + b1_ref[...])
        h_ref[...] = jnp.concatenate(
            [h, jnp.ones((B, 1), jnp.float32),
             jnp.zeros((B, KA - HID - 1), jnp.float32)],
            axis=1).astype(jnp.bfloat16)
        hn_ref[...] = jnp.sqrt(jnp.sum(h * h, axis=1, keepdims=True))
        m_ref[...] = jnp.full((B, 1), -jnp.inf, jnp.float32)
        acc_ref[...] = jnp.zeros((B, CH), jnp.float32)

    wf = w2a_ref[...].astype(jnp.float32)
    lane = lax.broadcasted_iota(jnp.int32, (VB, KA), 1)
    n2 = jnp.sum(jnp.where(lane < HID, wf * wf, 0.0), axis=1, keepdims=True)
    c2 = jnp.max(n2)
    bmax = jnp.max(wf[:, HID])
    m_old = m_ref[...]
    m_new = jnp.maximum(m_old, hn_ref[...] * jnp.sqrt(c2) + bmax)
    scale = jnp.exp(m_old - m_new)
    m_ref[...] = m_new

    ha = h_ref[...]
    sacc = acc_ref[...] * scale
    for c in range(NCH):
        wc = w2a_ref[pl.ds(c * CH, CH), :]
        l = lax.dot_general(ha, wc, (((1,), (1,)), ((), ())),
                            preferred_element_type=jnp.float32)
        sacc = sacc + jnp.exp(l - m_new)
    acc_ref[...] = sacc

    @pl.when(v == NV - 1)
    def _fin():
        s = jnp.sum(acc_ref[...], axis=1, keepdims=True)
        z_ref[...] = m_ref[...] + jnp.log(s)


def _out_body(h_ref, z_ref, w2a_ref, o_ref):
    ha = h_ref[...]
    z = z_ref[...]
    for c in range(NCH):
        wc = w2a_ref[pl.ds(c * CH, CH), :]
        l = lax.dot_general(ha, wc, (((1,), (1,)), ((), ())),
                            preferred_element_type=jnp.float32)
        o_ref[:, pl.ds(c * CH, CH)] = jnp.exp(l - z)


def _dense(h0, W1, b1, W2, b2):
    b1r = b1.reshape(1, HID)
    bias = jnp.concatenate(
        [b2, jnp.full((VP - V,), -1e30, jnp.float32)])[:, None]
    W2a = jnp.pad(
        jnp.concatenate([jnp.pad(W2, ((0, VP - V), (0, 0))), bias], axis=1),
        ((0, 0), (0, KA - HID - 1))).astype(jnp.bfloat16)

    h, z, _ = pl.pallas_call(
        _stats_body,
        grid=(NV,),
        in_specs=[
            pl.BlockSpec((B, FEAT), lambda v: (0, 0)),
            pl.BlockSpec((HID, FEAT), lambda v: (0, 0)),
            pl.BlockSpec((1, HID), lambda v: (0, 0)),
            pl.BlockSpec((VB, KA), lambda v: (v, 0)),
        ],
        out_specs=[
            pl.BlockSpec((B, KA), lambda v: (0, 0)),
            pl.BlockSpec((B, 1), lambda v: (0, 0)),
            pl.BlockSpec((B, 1), lambda v: (0, 0)),
        ],
        out_shape=[
            jax.ShapeDtypeStruct((B, KA), jnp.bfloat16),
            jax.ShapeDtypeStruct((B, 1), jnp.float32),
            jax.ShapeDtypeStruct((B, 1), jnp.float32),
        ],
        scratch_shapes=[
            pltpu.VMEM((B, 1), jnp.float32),
            pltpu.VMEM((B, CH), jnp.float32),
        ],
    )(h0, W1, b1r, W2a)

    out = pl.pallas_call(
        _out_body,
        grid=(NV,),
        in_specs=[
            pl.BlockSpec((B, KA), lambda v: (0, 0)),
            pl.BlockSpec((B, 1), lambda v: (0, 0)),
            pl.BlockSpec((VB, KA), lambda v: (v, 0)),
        ],
        out_specs=pl.BlockSpec((B, VB), lambda v: (0, v)),
        out_shape=jax.ShapeDtypeStruct((B, V), jnp.float32),
    )(h, z, W2a)
    return out


def kernel(x, emb, W1, b1, W2, b2):
    h0 = _sc_gather(emb, x.reshape(-1)).reshape(B, FEAT)
    return _dense(h0, W1, b1, W2, b2)
